# baseline (speedup 1.0000x reference)
import jax
import jax.numpy as jnp
from jax.experimental import pallas as pl
from jax.experimental.pallas import tpu as pltpu

_ROWS = 2048
_COLS = 1024
_BLK = 256
_N = _ROWS // _BLK
_CH = 8
_NCH = _BLK // _CH


def _fold(v):
    acc = v[:, 0:128]
    for j in range(1, 8):
        acc = acc + v[:, j * 128:(j + 1) * 128]
    return acc


def _loss_kernel(x_ref, y_ref, o_ref, a_ref, b_ref, acc_ref):
    i = pl.program_id(0)

    @pl.when(i == 0)
    def _init():
        acc_ref[...] = jnp.zeros((40, 128), jnp.float32)

    base = i * _BLK
    s_m = jnp.zeros((_CH, 128), jnp.float32)
    s_a = jnp.zeros((_CH, 128), jnp.float32)
    s_b = jnp.zeros((_CH, 128), jnp.float32)
    s_ab = jnp.zeros((_CH, 128), jnp.float32)
    s_bb = jnp.zeros((_CH, 128), jnp.float32)
    for k in range(_NCH):
        r = k * _CH
        xs = x_ref[r:r + _CH, :]
        ys = y_ref[r:r + _CH, :]
        disp = 1.0 / jnp.maximum(xs, 1e-6)
        mask = (xs > 0.1) & (xs < 100.0)
        zero = jnp.zeros_like(xs)
        a = jnp.where(mask, disp, zero)
        b = jnp.where(mask, ys, zero)
        mf = jnp.where(mask, 1.0, zero)
        a_ref[pl.ds(base + r, _CH), :] = a
        b_ref[pl.ds(base + r, _CH), :] = b
        s_m = s_m + _fold(mf)
        s_a = s_a + _fold(a)
        s_b = s_b + _fold(b)
        s_ab = s_ab + _fold(a * b)
        s_bb = s_bb + _fold(b * b)
    acc_ref[0:8, :] += s_m
    acc_ref[8:16, :] += s_a
    acc_ref[16:24, :] += s_b
    acc_ref[24:32, :] += s_ab
    acc_ref[32:40, :] += s_bb

    @pl.when(i == _N - 1)
    def _finish():
        cnt = jnp.maximum(jnp.sum(acc_ref[0:8, :]), 1.0)
        mean_r = jnp.sum(acc_ref[8:16, :]) / cnt
        mean_p = jnp.sum(acc_ref[16:24, :]) / cnt
        mean_rp = jnp.sum(acc_ref[24:32, :]) / cnt
        mean_pp = jnp.sum(acc_ref[32:40, :]) / cnt
        covar = mean_rp - mean_r * mean_p
        var_p = mean_pp - mean_p * mean_p
        s = jnp.maximum(covar / (var_p + 1e-8), 1e-4)
        t = mean_r - s * mean_p

        def body(k, l_acc):
            af = a_ref[pl.ds(k * _CH, _CH), :]
            bf = b_ref[pl.ds(k * _CH, _CH), :]
            tm = jnp.where(af > 0.0, t, 0.0)
            return l_acc + _fold(jnp.abs(af - s * bf - tm))

        l_acc = jax.lax.fori_loop(
            0, _ROWS // _CH, body, jnp.zeros((_CH, 128), jnp.float32))
        o_ref[...] = jnp.full((1, 1), jnp.sum(l_acc) / cnt, jnp.float32)


def kernel(render_depth, prior_disp):
    x = render_depth.reshape(_ROWS, _COLS)
    y = prior_disp.reshape(_ROWS, _COLS)

    out = pl.pallas_call(
        _loss_kernel,
        grid=(_N,),
        in_specs=[
            pl.BlockSpec((_BLK, _COLS), lambda i: (i, 0)),
            pl.BlockSpec((_BLK, _COLS), lambda i: (i, 0)),
        ],
        out_specs=pl.BlockSpec((1, 1), lambda i: (0, 0)),
        out_shape=jax.ShapeDtypeStruct((1, 1), jnp.float32),
        scratch_shapes=[
            pltpu.VMEM((_ROWS, _COLS), jnp.float32),
            pltpu.VMEM((_ROWS, _COLS), jnp.float32),
            pltpu.VMEM((40, 128), jnp.float32),
        ],
    )(x, y)
    return out.reshape(())


# speedup vs baseline: 2.1184x; 2.1184x over previous
import jax
import jax.numpy as jnp
from jax.experimental import pallas as pl
from jax.experimental.pallas import tpu as pltpu

_B = 8
_H = 512
_W = 512
_CH = 8
_NCH = _H // _CH


def _fold(v):
    acc = v[:, 0:128]
    for j in range(1, 4):
        acc = acc + v[:, j * 128:(j + 1) * 128]
    return acc


def _loss_kernel(x_ref, y_ref, o_ref, a_ref, b_ref, acc_ref):
    i = pl.program_id(0)

    @pl.when(i == 0)
    def _init():
        acc_ref[...] = jnp.zeros((40, 128), jnp.float32)

    base = i * _H
    s_m = jnp.zeros((_CH, 128), jnp.float32)
    s_a = jnp.zeros((_CH, 128), jnp.float32)
    s_b = jnp.zeros((_CH, 128), jnp.float32)
    s_ab = jnp.zeros((_CH, 128), jnp.float32)
    s_bb = jnp.zeros((_CH, 128), jnp.float32)
    for k in range(_NCH):
        r = k * _CH
        xs = x_ref[0, 0, r:r + _CH, :]
        ys = y_ref[0, 0, r:r + _CH, :]
        disp = 1.0 / jnp.maximum(xs, 1e-6)
        mask = (xs > 0.1) & (xs < 100.0)
        zero = jnp.zeros_like(xs)
        a = jnp.where(mask, disp, zero)
        b = jnp.where(mask, ys, zero)
        mf = jnp.where(mask, 1.0, zero)
        a_ref[pl.ds(base + r, _CH), :] = a
        b_ref[pl.ds(base + r, _CH), :] = b
        s_m = s_m + _fold(mf)
        s_a = s_a + _fold(a)
        s_b = s_b + _fold(b)
        s_ab = s_ab + _fold(a * b)
        s_bb = s_bb + _fold(b * b)
    acc_ref[0:8, :] += s_m
    acc_ref[8:16, :] += s_a
    acc_ref[16:24, :] += s_b
    acc_ref[24:32, :] += s_ab
    acc_ref[32:40, :] += s_bb

    @pl.when(i == _B - 1)
    def _finish():
        cnt = jnp.maximum(jnp.sum(acc_ref[0:8, :]), 1.0)
        mean_r = jnp.sum(acc_ref[8:16, :]) / cnt
        mean_p = jnp.sum(acc_ref[16:24, :]) / cnt
        mean_rp = jnp.sum(acc_ref[24:32, :]) / cnt
        mean_pp = jnp.sum(acc_ref[32:40, :]) / cnt
        covar = mean_rp - mean_r * mean_p
        var_p = mean_pp - mean_p * mean_p
        s = jnp.maximum(covar / (var_p + 1e-8), 1e-4)
        t = mean_r - s * mean_p

        def body(k, l_acc):
            af = a_ref[pl.ds(k * _CH, _CH), :]
            bf = b_ref[pl.ds(k * _CH, _CH), :]
            tm = jnp.where(af > 0.0, t, 0.0)
            return l_acc + _fold(jnp.abs(af - s * bf - tm))

        l_acc = jax.lax.fori_loop(
            0, (_B * _H) // _CH, body, jnp.zeros((_CH, 128), jnp.float32))
        o_ref[...] = jnp.full((1, 1), jnp.sum(l_acc) / cnt, jnp.float32)


def kernel(render_depth, prior_disp):
    out = pl.pallas_call(
        _loss_kernel,
        grid=(_B,),
        in_specs=[
            pl.BlockSpec((1, 1, _H, _W), lambda i: (i, 0, 0, 0)),
            pl.BlockSpec((1, 1, _H, _W), lambda i: (i, 0, 0, 0)),
        ],
        out_specs=pl.BlockSpec((1, 1), lambda i: (0, 0)),
        out_shape=jax.ShapeDtypeStruct((1, 1), jnp.float32),
        scratch_shapes=[
            pltpu.VMEM((_B * _H, _W), jnp.float32),
            pltpu.VMEM((_B * _H, _W), jnp.float32),
            pltpu.VMEM((40, 128), jnp.float32),
        ],
    )(render_depth, prior_disp)
    return out.reshape(())
